# baseline (speedup 1.0000x reference)
import math

import jax
import jax.numpy as jnp
from jax import lax
from jax.experimental import pallas as pl
from jax.experimental.pallas import tpu as pltpu

_T, _D, _E, _P, _K = 8192, 2048, 64, 64, 8
_EPS = 1e-5
_TM = 1024
_NSPLIT = 4
_NC = _P * _E // _NSPLIT
_PREC = lax.Precision.DEFAULT
_G = _T // _TM


def _ln(v, g, b):
    mu = jnp.mean(v, axis=1, keepdims=True)
    var = jnp.mean((v - mu) ** 2, axis=1, keepdims=True)
    return (v - mu) / jnp.sqrt(var + _EPS) * g + b


def _fold(sq):
    width = sq.shape[1]
    while width > _E:
        width //= 2
        sq = sq[:, :width] + sq[:, width:]
    return sq


def _body(x_ref, wg_ref, wp0_ref, wp1_ref, wp2_ref, wp3_ref, par_ref,
          w_out, i_out, s_out, r_scr, q_scr):
    r_prev = r_scr[...]
    q_prev = q_scr[...]

    x = x_ref[...]
    r = lax.dot_general(
        x, wg_ref[...], (((1,), (0,)), ((), ())),
        preferred_element_type=jnp.float32, precision=_PREC)
    ssq = None
    for wp_ref in (wp0_ref, wp1_ref, wp2_ref, wp3_ref):
        pj = lax.dot_general(
            x, wp_ref[...], (((1,), (0,)), ((), ())),
            preferred_element_type=jnp.float32, precision=_PREC)
        part = _fold(pj * pj)
        ssq = part if ssq is None else ssq + part
    r_scr[...] = r
    q_scr[...] = ssq

    c = jnp.sqrt(q_prev) * (1.0 / math.sqrt(_P))

    gamma_r = par_ref[0:1, :]
    beta_r = par_ref[1:2, :]
    gamma_c = par_ref[2:3, :]
    beta_c = par_ref[3:4, :]
    alpha = par_ref[4:5, :]
    gate = 1.0 / (1.0 + jnp.exp(-alpha))

    s = _ln(r_prev, gamma_r, beta_r) + gate * _ln(c, gamma_c, beta_c)
    s_out[...] = s

    m = jnp.max(s, axis=1, keepdims=True)
    p = jnp.exp(s - m)
    w = p / jnp.sum(p, axis=1, keepdims=True)

    iota = lax.broadcasted_iota(jnp.int32, (_TM, _E), 1)
    key = (lax.bitcast_convert_type(w, jnp.int32) & ~63) | (_E - 1 - iota)
    vals, idxs = [], []
    for _ in range(_K):
        mx = jnp.max(key, axis=1, keepdims=True)
        idxs.append((_E - 1) - (mx & 63))
        vals.append(lax.bitcast_convert_type(mx & ~63, jnp.float32))
        key = jnp.where(key == mx, 0, key)
    topw = jnp.concatenate(vals, axis=1)
    topi = jnp.concatenate(idxs, axis=1)
    w_out[...] = topw / jnp.sum(topw, axis=1, keepdims=True)
    i_out[...] = topi


def kernel(hidden_states, W_g, W_probe, alpha, gamma_r, beta_r, gamma_c, beta_c):
    x = hidden_states.astype(jnp.bfloat16)
    wg_t = W_g.astype(jnp.bfloat16).T
    wp_t = W_probe.astype(jnp.bfloat16).transpose(2, 1, 0).reshape(_D, _P * _E)
    wps = [wp_t[:, j:j + _NC] for j in range(0, _P * _E, _NC)]
    params = jnp.concatenate(
        [gamma_r[None, :], beta_r[None, :], gamma_c[None, :], beta_c[None, :],
         jnp.full((1, _E), alpha, jnp.float32), jnp.zeros((3, _E), jnp.float32)],
        axis=0)

    last = _G - 1
    outs = pl.pallas_call(
        _body,
        grid=(_G + 1,),
        in_specs=[
            pl.BlockSpec((_TM, _D), lambda i: (jnp.minimum(i, last), 0)),
            pl.BlockSpec((_D, _E), lambda i: (0, 0)),
        ] + [pl.BlockSpec((_D, _NC), lambda i: (0, 0))] * _NSPLIT + [
            pl.BlockSpec((8, _E), lambda i: (0, 0)),
        ],
        out_specs=[
            pl.BlockSpec((_TM, _K), lambda i: (jnp.maximum(i - 1, 0), 0)),
            pl.BlockSpec((_TM, _K), lambda i: (jnp.maximum(i - 1, 0), 0)),
            pl.BlockSpec((_TM, _E), lambda i: (jnp.maximum(i - 1, 0), 0)),
        ],
        out_shape=[
            jax.ShapeDtypeStruct((_T, _K), jnp.float32),
            jax.ShapeDtypeStruct((_T, _K), jnp.int32),
            jax.ShapeDtypeStruct((_T, _E), jnp.float32),
        ],
        scratch_shapes=[
            pltpu.VMEM((_TM, _E), jnp.float32),
            pltpu.VMEM((_TM, _E), jnp.float32),
        ],
    )(x, wg_t, *wps, params)
    return outs[0].astype(hidden_states.dtype), outs[1], outs[2]


# speedup vs baseline: 1.1313x; 1.1313x over previous
import math

import jax
import jax.numpy as jnp
from jax import lax
from jax.experimental import pallas as pl
from jax.experimental.pallas import tpu as pltpu

_T, _D, _E, _P, _K = 8192, 2048, 64, 64, 8
_EPS = 1e-5
_TM = 1024
_NSPLIT = 4
_NC = _P * _E // _NSPLIT
_PREC = lax.Precision.DEFAULT
_G = _T // _TM


def _ln(v, g, b):
    mu = jnp.mean(v, axis=1, keepdims=True)
    var = jnp.mean((v - mu) ** 2, axis=1, keepdims=True)
    return (v - mu) / jnp.sqrt(var + _EPS) * g + b


def _fold(sq):
    width = sq.shape[1]
    while width > _E:
        width //= 2
        sq = sq[:, :width] + sq[:, width:]
    return sq


def _body(x_ref, wg_ref, wp0_ref, wp1_ref, wp2_ref, wp3_ref, par_ref,
          w_out, i_out, s_out, r_scr, q_scr):
    r_prev = r_scr[...]
    q_prev = q_scr[...]

    x = x_ref[...].astype(jnp.bfloat16)
    r = lax.dot_general(
        x, wg_ref[...], (((1,), (0,)), ((), ())),
        preferred_element_type=jnp.float32, precision=_PREC)
    ssq = None
    for wp_ref in (wp0_ref, wp1_ref, wp2_ref, wp3_ref):
        pj = lax.dot_general(
            x, wp_ref[...], (((1,), (0,)), ((), ())),
            preferred_element_type=jnp.float32, precision=_PREC)
        part = _fold(pj * pj)
        ssq = part if ssq is None else ssq + part
    r_scr[...] = r
    q_scr[...] = ssq

    c = jnp.sqrt(q_prev) * (1.0 / math.sqrt(_P))

    gamma_r = par_ref[0:1, :]
    beta_r = par_ref[1:2, :]
    gamma_c = par_ref[2:3, :]
    beta_c = par_ref[3:4, :]
    alpha = par_ref[4:5, :]
    gate = 1.0 / (1.0 + jnp.exp(-alpha))

    s = _ln(r_prev, gamma_r, beta_r) + gate * _ln(c, gamma_c, beta_c)
    s_out[...] = s

    m = jnp.max(s, axis=1, keepdims=True)
    p = jnp.exp(s - m)
    w = p / jnp.sum(p, axis=1, keepdims=True)

    iota = lax.broadcasted_iota(jnp.int32, (_TM, _E), 1)
    key = (lax.bitcast_convert_type(w, jnp.int32) & ~63) | (_E - 1 - iota)
    vals, idxs = [], []
    for _ in range(_K):
        mx = jnp.max(key, axis=1, keepdims=True)
        idxs.append((_E - 1) - (mx & 63))
        vals.append(lax.bitcast_convert_type(mx & ~63, jnp.float32))
        key = jnp.where(key == mx, 0, key)
    topw = jnp.concatenate(vals, axis=1)
    topi = jnp.concatenate(idxs, axis=1)
    w_out[...] = topw / jnp.sum(topw, axis=1, keepdims=True)
    i_out[...] = topi


def kernel(hidden_states, W_g, W_probe, alpha, gamma_r, beta_r, gamma_c, beta_c):
    x = hidden_states.astype(jnp.float32)
    wg_t = W_g.astype(jnp.bfloat16).T
    wp_t = W_probe.astype(jnp.bfloat16).transpose(2, 1, 0).reshape(_D, _P * _E)
    wps = [wp_t[:, j:j + _NC] for j in range(0, _P * _E, _NC)]
    params = jnp.concatenate(
        [gamma_r[None, :], beta_r[None, :], gamma_c[None, :], beta_c[None, :],
         jnp.full((1, _E), alpha, jnp.float32), jnp.zeros((3, _E), jnp.float32)],
        axis=0)

    last = _G - 1
    outs = pl.pallas_call(
        _body,
        grid=(_G + 1,),
        in_specs=[
            pl.BlockSpec((_TM, _D), lambda i: (jnp.minimum(i, last), 0)),
            pl.BlockSpec((_D, _E), lambda i: (0, 0)),
        ] + [pl.BlockSpec((_D, _NC), lambda i: (0, 0))] * _NSPLIT + [
            pl.BlockSpec((8, _E), lambda i: (0, 0)),
        ],
        out_specs=[
            pl.BlockSpec((_TM, _K), lambda i: (jnp.maximum(i - 1, 0), 0)),
            pl.BlockSpec((_TM, _K), lambda i: (jnp.maximum(i - 1, 0), 0)),
            pl.BlockSpec((_TM, _E), lambda i: (jnp.maximum(i - 1, 0), 0)),
        ],
        out_shape=[
            jax.ShapeDtypeStruct((_T, _K), jnp.float32),
            jax.ShapeDtypeStruct((_T, _K), jnp.int32),
            jax.ShapeDtypeStruct((_T, _E), jnp.float32),
        ],
        scratch_shapes=[
            pltpu.VMEM((_TM, _E), jnp.float32),
            pltpu.VMEM((_TM, _E), jnp.float32),
        ],
    )(x, wg_t, *wps, params)
    return outs[0].astype(hidden_states.dtype), outs[1], outs[2]


# speedup vs baseline: 1.2122x; 1.0716x over previous
import math

import jax
import jax.numpy as jnp
from jax import lax
from jax.experimental import pallas as pl
from jax.experimental.pallas import tpu as pltpu

_T, _D, _E, _P, _K = 8192, 2048, 64, 64, 8
_EPS = 1e-5
_TM = 512
_PREC = lax.Precision.DEFAULT
_G = _T // _TM


def _ln(v, g, b):
    mu = jnp.mean(v, axis=1, keepdims=True)
    var = jnp.mean((v - mu) ** 2, axis=1, keepdims=True)
    return (v - mu) / jnp.sqrt(var + _EPS) * g + b


def _fold(sq):
    width = sq.shape[1]
    while width > _E:
        width //= 2
        sq = sq[:, :width] + sq[:, width:]
    return sq


def _body(x_ref, wg_ref, wpn_ref, par_ref, w_out, i_out, s_out,
          wt_scr, r_scr, q_scr):
    i = pl.program_id(0)

    @pl.when(i == 0)
    def _():
        wt_scr[...] = wpn_ref[...].T

    r_prev = r_scr[...]
    q_prev = q_scr[...]

    x = x_ref[...].astype(jnp.bfloat16)
    r = lax.dot_general(
        x, wg_ref[...], (((1,), (0,)), ((), ())),
        preferred_element_type=jnp.float32, precision=_PREC)
    pj = lax.dot_general(
        x, wt_scr[...], (((1,), (0,)), ((), ())),
        preferred_element_type=jnp.float32, precision=_PREC)
    ssq = _fold(pj * pj)
    r_scr[...] = r
    q_scr[...] = ssq

    c = jnp.sqrt(q_prev) * (1.0 / math.sqrt(_P))

    gamma_r = par_ref[0:1, :]
    beta_r = par_ref[1:2, :]
    gamma_c = par_ref[2:3, :]
    beta_c = par_ref[3:4, :]
    alpha = par_ref[4:5, :]
    gate = 1.0 / (1.0 + jnp.exp(-alpha))

    s = _ln(r_prev, gamma_r, beta_r) + gate * _ln(c, gamma_c, beta_c)
    s_out[...] = s

    m = jnp.max(s, axis=1, keepdims=True)
    p = jnp.exp(s - m)
    w = p / jnp.sum(p, axis=1, keepdims=True)

    iota = lax.broadcasted_iota(jnp.int32, (_TM, _E), 1)
    key = (lax.bitcast_convert_type(w, jnp.int32) & ~63) | (_E - 1 - iota)
    vals, idxs = [], []
    for _ in range(_K):
        mx = jnp.max(key, axis=1, keepdims=True)
        idxs.append((_E - 1) - (mx & 63))
        vals.append(lax.bitcast_convert_type(mx & ~63, jnp.float32))
        key = jnp.where(key == mx, 0, key)
    topw = jnp.concatenate(vals, axis=1)
    topi = jnp.concatenate(idxs, axis=1)
    w_out[...] = topw / jnp.sum(topw, axis=1, keepdims=True)
    i_out[...] = topi


def kernel(hidden_states, W_g, W_probe, alpha, gamma_r, beta_r, gamma_c, beta_c):
    x = hidden_states.astype(jnp.float32)
    wg_t = W_g.astype(jnp.bfloat16).T
    wpn = W_probe.astype(jnp.bfloat16).transpose(1, 0, 2).reshape(_P * _E, _D)
    params = jnp.concatenate(
        [gamma_r[None, :], beta_r[None, :], gamma_c[None, :], beta_c[None, :],
         jnp.full((1, _E), alpha, jnp.float32), jnp.zeros((3, _E), jnp.float32)],
        axis=0)

    last = _G - 1
    outs = pl.pallas_call(
        _body,
        grid=(_G + 1,),
        in_specs=[
            pl.BlockSpec((_TM, _D), lambda i: (jnp.minimum(i, last), 0)),
            pl.BlockSpec((_D, _E), lambda i: (0, 0)),
            pl.BlockSpec((_P * _E, _D), lambda i: (0, 0)),
            pl.BlockSpec((8, _E), lambda i: (0, 0)),
        ],
        out_specs=[
            pl.BlockSpec((_TM, _K), lambda i: (jnp.maximum(i - 1, 0), 0)),
            pl.BlockSpec((_TM, _K), lambda i: (jnp.maximum(i - 1, 0), 0)),
            pl.BlockSpec((_TM, _E), lambda i: (jnp.maximum(i - 1, 0), 0)),
        ],
        out_shape=[
            jax.ShapeDtypeStruct((_T, _K), jnp.float32),
            jax.ShapeDtypeStruct((_T, _K), jnp.int32),
            jax.ShapeDtypeStruct((_T, _E), jnp.float32),
        ],
        scratch_shapes=[
            pltpu.VMEM((_D, _P * _E), jnp.bfloat16),
            pltpu.VMEM((_TM, _E), jnp.float32),
            pltpu.VMEM((_TM, _E), jnp.float32),
        ],
    )(x, wg_t, wpn, params)
    return outs[0].astype(hidden_states.dtype), outs[1], outs[2]
